# baseline (speedup 1.0000x reference)
import functools

import jax
import jax.numpy as jnp
from jax import lax
from jax.experimental import pallas as pl
from jax.experimental.pallas import tpu as pltpu
from jax.experimental.pallas import tpu_sc as plsc

B = 16384
S = 200
D = 64
NC = 2
NS = 16
NW = NC * NS
BPW = B // NW
CB = 4
NCH = BPW // CB
G = 80
NG = (CB * S) // G
L = 16
NVR = D // L

_mesh = plsc.VectorSubcoreMesh(core_axis_name="c", subcore_axis_name="s")

V = 1000000
TBLK = 128
NBLK = V // TBLK
VTAIL = NBLK * TBLK
BLK_PER_W = NBLK // NW
BLK_REM = NBLK % NW


@functools.partial(
    pl.kernel,
    mesh=_mesh,
    out_type=jax.ShapeDtypeStruct((V * D,), jnp.float32),
    scratch_types=[
        pltpu.VMEM((2 * D, TBLK), jnp.float32),
        pltpu.VMEM((2 * TBLK * D,), jnp.float32),
        pltpu.VMEM(((V - VTAIL) * D,), jnp.float32),
        pltpu.SemaphoreType.DMA((2,)),
        pltpu.SemaphoreType.DMA((2,)),
    ],
    compiler_params=pltpu.CompilerParams(
        use_tc_tiling_on_sc=True, needs_layout_passes=False
    ),
)
def _table_lin(tt_hbm, tail_hbm, out_hbm, slab_v, tslab_v, tail_v, in_sem, out_sem):
    wid = lax.axis_index("s") * NC + lax.axis_index("c")
    base = wid * BLK_PER_W + jnp.minimum(wid, BLK_REM)
    NPAIR = BLK_PER_W // 2
    lane16 = lax.broadcasted_iota(jnp.int32, (L,), 0)
    cm = [lane16 + L * m for m in range(TBLK // L)]
    c2 = [
        [(lane16 + L * m) * D + s * TBLK * D for m in range(TBLK // L)]
        for s in range(2)
    ]

    def in_copy(c, s):
        return pltpu.make_async_copy(
            tt_hbm.at[:, pl.ds(c * TBLK, TBLK)],
            slab_v.at[pl.ds(s * D, D), :],
            in_sem.at[s],
        )

    def out_copy(c, s):
        return pltpu.make_async_copy(
            tslab_v.at[pl.ds(s * TBLK * D, TBLK * D)],
            out_hbm.at[pl.ds(c * (TBLK * D), TBLK * D)],
            out_sem.at[s],
        )

    def transpose_slot(s):
        @plsc.parallel_loop(0, D, unroll=4)
        def tf(f0):
            fq = jnp.bitwise_and(f0 + lane16, D - 1)
            frow = fq + s * D
            for m in range(TBLK // L):
                val = plsc.load_gather(slab_v, [frow, cm[m]])
                plsc.store_scatter(tslab_v, [c2[s][m] + fq], val)

    in_copy(base, 0).start()

    def pair_body(t, carry):
        c0 = base + 2 * t

        in_copy(c0 + 1, 1).start()
        in_copy(c0, 0).wait()

        @pl.when(t >= 1)
        def _():
            out_copy(c0 - 2, 0).wait()

        transpose_slot(0)
        out_copy(c0, 0).start()

        @pl.when(t + 1 < NPAIR)
        def _():
            in_copy(c0 + 2, 0).start()

        in_copy(c0 + 1, 1).wait()

        @pl.when(t >= 1)
        def _():
            out_copy(c0 - 1, 1).wait()

        transpose_slot(1)
        out_copy(c0 + 1, 1).start()
        return carry

    lax.fori_loop(0, NPAIR, pair_body, 0)

    out_copy(base + 2 * NPAIR - 2, 0).wait()
    out_copy(base + 2 * NPAIR - 1, 1).wait()

    @pl.when(wid < BLK_REM)
    def _():
        c = base + 2 * NPAIR
        in_copy(c, 0).start()
        in_copy(c, 0).wait()
        transpose_slot(0)
        out_copy(c, 0).start()
        out_copy(c, 0).wait()

    @pl.when(wid == 0)
    def _():
        pltpu.sync_copy(tail_hbm, tail_v)
        pltpu.sync_copy(tail_v, out_hbm.at[pl.ds(VTAIL * D, (V - VTAIL) * D)])


@functools.partial(
    pl.kernel,
    mesh=_mesh,
    out_type=jax.ShapeDtypeStruct((B, D), jnp.float32),
    scratch_types=[
        pltpu.VMEM((3, CB * S), jnp.int32),
        pltpu.VMEM((2, CB * S, D), jnp.float32),
        pltpu.VMEM((2, CB, D), jnp.float32),
        pltpu.SemaphoreType.DMA((3,)),
        pltpu.SemaphoreType.DMA((2,)),
        pltpu.SemaphoreType.DMA((2,)),
    ],
    compiler_params=pltpu.CompilerParams(use_tc_tiling_on_sc=False),
)
def _emb_pool(
    x_hbm, table_hbm, out_hbm, idx_v, rows_v, out_v, idx_sem, gat_sem, out_sem
):
    wid = lax.axis_index("s") * NC + lax.axis_index("c")
    lane = lax.broadcasted_iota(jnp.int32, (L,), 0)
    lo8 = jnp.minimum(jnp.maximum(8 - lane, 0), 1)
    hi8 = 1 - lo8

    def idx_copy(c, sl):
        return pltpu.make_async_copy(
            x_hbm.at[pl.ds((wid * BPW + c * CB) * S, CB * S)],
            idx_v.at[sl],
            idx_sem.at[sl],
        )

    def fire_gathers(bsl, gsl):
        for j in range(NG):
            pltpu.async_copy(
                table_hbm.at[idx_v.at[bsl, pl.ds(j * G, G)]],
                rows_v.at[gsl, pl.ds(j * G, G)],
                gat_sem.at[gsl],
            )

    def wait_gathers(bsl, gsl):
        for j in range(NG):
            pltpu.make_async_copy(
                table_hbm.at[idx_v.at[bsl, pl.ds(j * G, G)]],
                rows_v.at[gsl, pl.ds(j * G, G)],
                gat_sem.at[gsl],
            ).wait()

    idx_copy(0, 0).start()
    idx_copy(0, 0).wait()
    fire_gathers(0, 0)
    idx_copy(1, 1).start()

    def chunk_body(i, carry):
        row0 = wid * BPW + i * CB
        cur = jnp.bitwise_and(i, 1)
        nxt = jnp.bitwise_and(i + 1, 1)
        bsl = lax.rem(i, 3)
        bsl1 = lax.rem(i + 1, 3)
        bsl2 = lax.rem(i + 2, 3)

        @pl.when(i + 1 < NCH)
        def _():
            idx_copy(i + 1, bsl1).wait()
            fire_gathers(bsl1, nxt)

        @pl.when(i + 2 < NCH)
        def _():
            idx_copy(i + 2, bsl2).start()

        invs = []
        for r in range(CB):
            if r % 2 == 0:
                full0 = (S * r) // L
                shared_k = full0 + 12
                shared_mask = lo8
            else:
                shared_k = (S * r - 8) // L
                full0 = shared_k + 1
                shared_mask = hi8
            ones = (
                jnp.minimum(idx_v[bsl, pl.ds(L * shared_k, L)], 1) * shared_mask
            )
            for k in range(full0, full0 + 12):
                ones = ones + jnp.minimum(idx_v[bsl, pl.ds(L * k, L)], 1)
            cnt = ones[0]
            for j in range(1, L):
                cnt = cnt + ones[j]
            denom = jnp.maximum(
                jnp.broadcast_to(cnt, (L,)).astype(jnp.float32), 1.0
            )
            invs.append(1.0 / denom)

        wait_gathers(bsl, cur)

        @pl.when(i >= 2)
        def _():
            pltpu.make_async_copy(
                out_v.at[cur],
                out_hbm.at[pl.ds(row0 - 2 * CB, CB)],
                out_sem.at[cur],
            ).wait()

        for r in range(CB):
            @plsc.parallel_loop(
                0,
                S,
                unroll=8,
                carry=tuple(jnp.zeros((L,), jnp.float32) for _ in range(NVR)),
            )
            def srow(s, accs):
                row = r * S + s
                return tuple(
                    a + rows_v[cur, row, pl.ds(L * l, L)]
                    for l, a in enumerate(accs)
                )

            accs = srow
            for l in range(NVR):
                out_v[cur, r, pl.ds(L * l, L)] = accs[l] * invs[r]

        pltpu.async_copy(
            out_v.at[cur], out_hbm.at[pl.ds(row0, CB)], out_sem.at[cur]
        )
        return carry

    lax.fori_loop(0, NCH, chunk_body, 0)

    for t in (NCH - 2, NCH - 1):
        pltpu.make_async_copy(
            out_v.at[t % 2],
            out_hbm.at[pl.ds(wid * BPW + t * CB, CB)],
            out_sem.at[t % 2],
        ).wait()


def kernel(x, table):
    tail = table[VTAIL:].reshape(-1)
    tlin = _table_lin(table.T, tail)
    return _emb_pool(x.reshape(-1), tlin.reshape(V, D))


# speedup vs baseline: 1.1309x; 1.1309x over previous
import functools

import jax
import jax.numpy as jnp
from jax import lax
from jax.experimental import pallas as pl
from jax.experimental.pallas import tpu as pltpu
from jax.experimental.pallas import tpu_sc as plsc

B = 16384
S = 200
D = 64
NC = 2
NS = 16
NW = NC * NS
BPW = B // NW
CB = 4
NCH = BPW // CB
G = 80
NG = (CB * S) // G
L = 16
NVR = D // L

_mesh = plsc.VectorSubcoreMesh(core_axis_name="c", subcore_axis_name="s")

V = 1000000
TBLK = 128
NBLK = V // TBLK
VTAIL = NBLK * TBLK
BLK_PER_W = NBLK // NW
BLK_REM = NBLK % NW


@functools.partial(
    pl.kernel,
    mesh=_mesh,
    out_type=jax.ShapeDtypeStruct((V * D,), jnp.float32),
    scratch_types=[
        pltpu.VMEM((2 * D, TBLK), jnp.float32),
        pltpu.VMEM((2 * TBLK * D,), jnp.float32),
        pltpu.VMEM(((V - VTAIL) * D,), jnp.float32),
        pltpu.SemaphoreType.DMA((2,)),
        pltpu.SemaphoreType.DMA((2,)),
    ],
    compiler_params=pltpu.CompilerParams(
        use_tc_tiling_on_sc=True, needs_layout_passes=False
    ),
)
def _table_lin(tt_hbm, tail_hbm, out_hbm, slab_v, tslab_v, tail_v, in_sem, out_sem):
    wid = lax.axis_index("s") * NC + lax.axis_index("c")
    base = wid * BLK_PER_W + jnp.minimum(wid, BLK_REM)
    NPAIR = BLK_PER_W // 2
    lane16 = lax.broadcasted_iota(jnp.int32, (L,), 0)
    cm = [lane16 + L * m for m in range(TBLK // L)]
    c2 = [
        [(lane16 + L * m) * D + s * TBLK * D for m in range(TBLK // L)]
        for s in range(2)
    ]

    def in_copy(c, s):
        return pltpu.make_async_copy(
            tt_hbm.at[:, pl.ds(c * TBLK, TBLK)],
            slab_v.at[pl.ds(s * D, D), :],
            in_sem.at[s],
        )

    def out_copy(c, s):
        return pltpu.make_async_copy(
            tslab_v.at[pl.ds(s * TBLK * D, TBLK * D)],
            out_hbm.at[pl.ds(c * (TBLK * D), TBLK * D)],
            out_sem.at[s],
        )

    def transpose_slot(s):
        @plsc.parallel_loop(0, D, unroll=4)
        def tf(f0):
            fq = jnp.bitwise_and(f0 + lane16, D - 1)
            frow = fq + s * D
            for m in range(TBLK // L):
                val = plsc.load_gather(slab_v, [frow, cm[m]])
                plsc.store_scatter(tslab_v, [c2[s][m] + fq], val)

    in_copy(base, 0).start()

    def pair_body(t, carry):
        c0 = base + 2 * t

        in_copy(c0 + 1, 1).start()
        in_copy(c0, 0).wait()

        @pl.when(t >= 1)
        def _():
            out_copy(c0 - 2, 0).wait()

        transpose_slot(0)
        out_copy(c0, 0).start()

        @pl.when(t + 1 < NPAIR)
        def _():
            in_copy(c0 + 2, 0).start()

        in_copy(c0 + 1, 1).wait()

        @pl.when(t >= 1)
        def _():
            out_copy(c0 - 1, 1).wait()

        transpose_slot(1)
        out_copy(c0 + 1, 1).start()
        return carry

    lax.fori_loop(0, NPAIR, pair_body, 0)

    out_copy(base + 2 * NPAIR - 2, 0).wait()
    out_copy(base + 2 * NPAIR - 1, 1).wait()

    @pl.when(wid < BLK_REM)
    def _():
        c = base + 2 * NPAIR
        in_copy(c, 0).start()
        in_copy(c, 0).wait()
        transpose_slot(0)
        out_copy(c, 0).start()
        out_copy(c, 0).wait()

    @pl.when(wid == 0)
    def _():
        pltpu.sync_copy(tail_hbm, tail_v)
        pltpu.sync_copy(tail_v, out_hbm.at[pl.ds(VTAIL * D, (V - VTAIL) * D)])


@functools.partial(
    pl.kernel,
    mesh=_mesh,
    out_type=jax.ShapeDtypeStruct((B, D), jnp.float32),
    scratch_types=[
        pltpu.VMEM((3, CB * S), jnp.int32),
        pltpu.VMEM((2, CB * S, D), jnp.float32),
        pltpu.VMEM((2, CB, D), jnp.float32),
        pltpu.SemaphoreType.DMA((3,)),
        pltpu.SemaphoreType.DMA((2,)),
        pltpu.SemaphoreType.DMA((2,)),
    ],
    compiler_params=pltpu.CompilerParams(use_tc_tiling_on_sc=False),
)
def _emb_pool(
    x_hbm, table_hbm, out_hbm, idx_v, rows_v, out_v, idx_sem, gat_sem, out_sem
):
    wid = lax.axis_index("s") * NC + lax.axis_index("c")
    lane = lax.broadcasted_iota(jnp.int32, (L,), 0)
    lo8 = jnp.minimum(jnp.maximum(8 - lane, 0), 1)
    hi8 = 1 - lo8

    def idx_copy(c, sl):
        return pltpu.make_async_copy(
            x_hbm.at[pl.ds((wid * BPW + c * CB) * S, CB * S)],
            idx_v.at[sl],
            idx_sem.at[sl],
        )

    def fire_gathers(bsl, gsl):
        for j in range(NG):
            pltpu.async_copy(
                table_hbm.at[idx_v.at[bsl, pl.ds(j * G, G)]],
                rows_v.at[gsl, pl.ds(j * G, G)],
                gat_sem.at[gsl],
            )

    def wait_gathers(bsl, gsl):
        for j in range(NG):
            pltpu.make_async_copy(
                table_hbm.at[idx_v.at[bsl, pl.ds(j * G, G)]],
                rows_v.at[gsl, pl.ds(j * G, G)],
                gat_sem.at[gsl],
            ).wait()

    idx_copy(0, 0).start()
    idx_copy(0, 0).wait()
    fire_gathers(0, 0)
    idx_copy(1, 1).start()

    def chunk_body(i, carry):
        row0 = wid * BPW + i * CB
        cur = jnp.bitwise_and(i, 1)
        nxt = jnp.bitwise_and(i + 1, 1)
        bsl = lax.rem(i, 3)
        bsl1 = lax.rem(i + 1, 3)
        bsl2 = lax.rem(i + 2, 3)

        @pl.when(i + 1 < NCH)
        def _():
            idx_copy(i + 1, bsl1).wait()
            fire_gathers(bsl1, nxt)

        @pl.when(i + 2 < NCH)
        def _():
            idx_copy(i + 2, bsl2).start()

        invs = []
        for r in range(CB):
            if r % 2 == 0:
                full0 = (S * r) // L
                shared_k = full0 + 12
                shared_mask = lo8
            else:
                shared_k = (S * r - 8) // L
                full0 = shared_k + 1
                shared_mask = hi8
            ones = (
                jnp.minimum(idx_v[bsl, pl.ds(L * shared_k, L)], 1) * shared_mask
            )
            for k in range(full0, full0 + 12):
                ones = ones + jnp.minimum(idx_v[bsl, pl.ds(L * k, L)], 1)
            cnt = ones[0]
            for j in range(1, L):
                cnt = cnt + ones[j]
            denom = jnp.maximum(
                jnp.broadcast_to(cnt, (L,)).astype(jnp.float32), 1.0
            )
            invs.append(1.0 / denom)

        wait_gathers(bsl, cur)

        @pl.when(i >= 2)
        def _():
            pltpu.make_async_copy(
                out_v.at[cur],
                out_hbm.at[pl.ds(row0 - 2 * CB, CB)],
                out_sem.at[cur],
            ).wait()

        for r in range(CB):
            def srow(s, accs):
                row = r * S + s
                return tuple(
                    a + rows_v[cur, row, pl.ds(L * l, L)]
                    for l, a in enumerate(accs)
                )

            accs = lax.fori_loop(
                0,
                S,
                srow,
                tuple(jnp.zeros((L,), jnp.float32) for _ in range(NVR)),
                unroll=10,
            )
            for l in range(NVR):
                out_v[cur, r, pl.ds(L * l, L)] = accs[l] * invs[r]

        pltpu.async_copy(
            out_v.at[cur], out_hbm.at[pl.ds(row0, CB)], out_sem.at[cur]
        )
        return carry

    lax.fori_loop(0, NCH, chunk_body, 0)

    for t in (NCH - 2, NCH - 1):
        pltpu.make_async_copy(
            out_v.at[t % 2],
            out_hbm.at[pl.ds(wid * BPW + t * CB, CB)],
            out_sem.at[t % 2],
        ).wait()


def kernel(x, table):
    tail = table[VTAIL:].reshape(-1)
    tlin = _table_lin(table.T, tail)
    return _emb_pool(x.reshape(-1), tlin.reshape(V, D))


# speedup vs baseline: 1.4309x; 1.2653x over previous
import functools

import jax
import jax.numpy as jnp
from jax import lax
from jax.experimental import pallas as pl
from jax.experimental.pallas import tpu as pltpu
from jax.experimental.pallas import tpu_sc as plsc

B = 16384
S = 200
D = 64
NC = 2
NS = 16
NW = NC * NS
BPW = B // NW
CB = 4
NCH = BPW // CB
G = 80
NG = (CB * S) // G
L = 16
NVR = D // L

_mesh = plsc.VectorSubcoreMesh(core_axis_name="c", subcore_axis_name="s")

V = 1000000
TBLK = 128
NBLK = V // TBLK
VTAIL = NBLK * TBLK
BLK_PER_W = NBLK // NW
BLK_REM = NBLK % NW


@functools.partial(
    pl.kernel,
    mesh=_mesh,
    out_type=jax.ShapeDtypeStruct((V * D // 2,), jnp.int32),
    scratch_types=[
        pltpu.VMEM((2 * D, TBLK), jnp.float32),
        pltpu.VMEM((2 * TBLK * D // 2,), jnp.int32),
        pltpu.VMEM(((V - VTAIL) * D // 2,), jnp.int32),
        pltpu.SemaphoreType.DMA((2,)),
        pltpu.SemaphoreType.DMA((2,)),
    ],
    compiler_params=pltpu.CompilerParams(
        use_tc_tiling_on_sc=True, needs_layout_passes=False
    ),
)
def _table_lin(tt_hbm, tail_hbm, out_hbm, slab_v, tslab_v, tail_v, in_sem, out_sem):
    wid = lax.axis_index("s") * NC + lax.axis_index("c")
    base = wid * BLK_PER_W + jnp.minimum(wid, BLK_REM)
    NPAIR = BLK_PER_W // 2
    lane16 = lax.broadcasted_iota(jnp.int32, (L,), 0)
    cm = [lane16 + L * m for m in range(TBLK // L)]
    DW = D // 2
    c2 = [
        [(lane16 + L * m) * DW + s * TBLK * DW for m in range(TBLK // L)]
        for s in range(2)
    ]

    def in_copy(c, s):
        return pltpu.make_async_copy(
            tt_hbm.at[:, pl.ds(c * TBLK, TBLK)],
            slab_v.at[pl.ds(s * D, D), :],
            in_sem.at[s],
        )

    def out_copy(c, s):
        return pltpu.make_async_copy(
            tslab_v.at[pl.ds(s * TBLK * DW, TBLK * DW)],
            out_hbm.at[pl.ds(c * (TBLK * DW), TBLK * DW)],
            out_sem.at[s],
        )

    def transpose_slot(s):
        @plsc.parallel_loop(0, DW, unroll=4)
        def tf(fp0):
            fpq = jnp.bitwise_and(fp0 + lane16, DW - 1)
            fe = 2 * fpq + s * D
            for m in range(TBLK // L):
                a = plsc.load_gather(slab_v, [fe, cm[m]])
                b = plsc.load_gather(slab_v, [fe + 1, cm[m]])
                w = plsc.bitcast(
                    plsc.pack(a, b, format=plsc.PackFormat.INTERLEAVED),
                    jnp.int32,
                )
                plsc.store_scatter(tslab_v, [c2[s][m] + fpq], w)

    in_copy(base, 0).start()

    def pair_body(t, carry):
        c0 = base + 2 * t

        in_copy(c0 + 1, 1).start()
        in_copy(c0, 0).wait()

        @pl.when(t >= 1)
        def _():
            out_copy(c0 - 2, 0).wait()

        transpose_slot(0)
        out_copy(c0, 0).start()

        @pl.when(t + 1 < NPAIR)
        def _():
            in_copy(c0 + 2, 0).start()

        in_copy(c0 + 1, 1).wait()

        @pl.when(t >= 1)
        def _():
            out_copy(c0 - 1, 1).wait()

        transpose_slot(1)
        out_copy(c0 + 1, 1).start()
        return carry

    lax.fori_loop(0, NPAIR, pair_body, 0)

    out_copy(base + 2 * NPAIR - 2, 0).wait()
    out_copy(base + 2 * NPAIR - 1, 1).wait()

    @pl.when(wid < BLK_REM)
    def _():
        c = base + 2 * NPAIR
        in_copy(c, 0).start()
        in_copy(c, 0).wait()
        transpose_slot(0)
        out_copy(c, 0).start()
        out_copy(c, 0).wait()

    @pl.when(wid == 0)
    def _():
        pltpu.sync_copy(tail_hbm, tail_v)
        pltpu.sync_copy(
            tail_v, out_hbm.at[pl.ds(VTAIL * (D // 2), (V - VTAIL) * (D // 2))]
        )


@functools.partial(
    pl.kernel,
    mesh=_mesh,
    out_type=jax.ShapeDtypeStruct((B * D,), jnp.float32),
    scratch_types=[
        pltpu.VMEM((3, CB * S), jnp.int32),
        pltpu.VMEM((2, CB * S, D // 2), jnp.int32),
        pltpu.VMEM((2 * CB * D,), jnp.float32),
        pltpu.SemaphoreType.DMA((3,)),
        pltpu.SemaphoreType.DMA((2,)),
        pltpu.SemaphoreType.DMA((2,)),
    ],
    compiler_params=pltpu.CompilerParams(
        use_tc_tiling_on_sc=False, needs_layout_passes=False
    ),
)
def _emb_pool(
    x_hbm, table_hbm, out_hbm, idx_v, rows_v, out_v, idx_sem, gat_sem, out_sem
):
    wid = lax.axis_index("s") * NC + lax.axis_index("c")
    lane = lax.broadcasted_iota(jnp.int32, (L,), 0)
    lo8 = jnp.minimum(jnp.maximum(8 - lane, 0), 1)
    hi8 = 1 - lo8

    def idx_copy(c, sl):
        return pltpu.make_async_copy(
            x_hbm.at[pl.ds((wid * BPW + c * CB) * S, CB * S)],
            idx_v.at[sl],
            idx_sem.at[sl],
        )

    def fire_gathers(bsl, gsl):
        for j in range(NG):
            pltpu.async_copy(
                table_hbm.at[idx_v.at[bsl, pl.ds(j * G, G)]],
                rows_v.at[gsl, pl.ds(j * G, G)],
                gat_sem.at[gsl],
            )

    def wait_gathers(bsl, gsl):
        for j in range(NG):
            pltpu.make_async_copy(
                table_hbm.at[idx_v.at[bsl, pl.ds(j * G, G)]],
                rows_v.at[gsl, pl.ds(j * G, G)],
                gat_sem.at[gsl],
            ).wait()

    idx_copy(0, 0).start()
    idx_copy(0, 0).wait()
    fire_gathers(0, 0)
    idx_copy(1, 1).start()

    def chunk_body(i, carry):
        row0 = wid * BPW + i * CB
        cur = jnp.bitwise_and(i, 1)
        nxt = jnp.bitwise_and(i + 1, 1)
        bsl = lax.rem(i, 3)
        bsl1 = lax.rem(i + 1, 3)
        bsl2 = lax.rem(i + 2, 3)

        @pl.when(i + 1 < NCH)
        def _():
            idx_copy(i + 1, bsl1).wait()
            fire_gathers(bsl1, nxt)

        @pl.when(i + 2 < NCH)
        def _():
            idx_copy(i + 2, bsl2).start()

        invs = []
        for r in range(CB):
            if r % 2 == 0:
                full0 = (S * r) // L
                shared_k = full0 + 12
                shared_mask = lo8
            else:
                shared_k = (S * r - 8) // L
                full0 = shared_k + 1
                shared_mask = hi8
            ones = (
                jnp.minimum(idx_v[bsl, pl.ds(L * shared_k, L)], 1) * shared_mask
            )
            for k in range(full0, full0 + 12):
                ones = ones + jnp.minimum(idx_v[bsl, pl.ds(L * k, L)], 1)
            cnt = ones[0]
            for j in range(1, L):
                cnt = cnt + ones[j]
            denom = jnp.maximum(
                jnp.broadcast_to(cnt, (L,)).astype(jnp.float32), 1.0
            )
            invs.append(1.0 / denom)

        wait_gathers(bsl, cur)

        @pl.when(i >= 2)
        def _():
            pltpu.make_async_copy(
                out_v.at[pl.ds(cur * (CB * D), CB * D)],
                out_hbm.at[pl.ds((row0 - 2 * CB) * D, CB * D)],
                out_sem.at[cur],
            ).wait()

        for r in range(CB):
            def srow(s, accs):
                row = r * S + s
                w0 = rows_v[cur, row, pl.ds(0, L)]
                w1 = rows_v[cur, row, pl.ds(L, L)]
                e0, o0 = plsc.unpack(
                    plsc.bitcast(w0, jnp.bfloat16),
                    format=plsc.PackFormat.INTERLEAVED,
                )
                e1, o1 = plsc.unpack(
                    plsc.bitcast(w1, jnp.bfloat16),
                    format=plsc.PackFormat.INTERLEAVED,
                )
                return (accs[0] + e0, accs[1] + o0, accs[2] + e1, accs[3] + o1)

            accs = lax.fori_loop(
                0,
                S,
                srow,
                tuple(jnp.zeros((L,), jnp.float32) for _ in range(NVR)),
                unroll=8,
            )
            obase = cur * (CB * D) + r * D + 2 * lane
            plsc.store_scatter(out_v, [obase], accs[0] * invs[r])
            plsc.store_scatter(out_v, [obase + 1], accs[1] * invs[r])
            plsc.store_scatter(out_v, [obase + D // 2], accs[2] * invs[r])
            plsc.store_scatter(out_v, [obase + D // 2 + 1], accs[3] * invs[r])

        pltpu.async_copy(
            out_v.at[pl.ds(cur * (CB * D), CB * D)],
            out_hbm.at[pl.ds(row0 * D, CB * D)],
            out_sem.at[cur],
        )
        return carry

    lax.fori_loop(0, NCH, chunk_body, 0)

    for t in (NCH - 2, NCH - 1):
        pltpu.make_async_copy(
            out_v.at[pl.ds((t % 2) * (CB * D), CB * D)],
            out_hbm.at[pl.ds((wid * BPW + t * CB) * D, CB * D)],
            out_sem.at[t % 2],
        ).wait()


def kernel(x, table):
    tail = lax.bitcast_convert_type(
        table[VTAIL:].astype(jnp.bfloat16).reshape(-1, 2), jnp.int32
    )
    tlin = _table_lin(table.T, tail)
    out = _emb_pool(x.reshape(-1), tlin.reshape(V, D // 2))
    return out.reshape(B, D)


# speedup vs baseline: 1.4420x; 1.0078x over previous
import functools

import jax
import jax.numpy as jnp
from jax import lax
from jax.experimental import pallas as pl
from jax.experimental.pallas import tpu as pltpu
from jax.experimental.pallas import tpu_sc as plsc

B = 16384
S = 200
D = 64
NC = 2
NS = 16
NW = NC * NS
BPW = B // NW
CB = 4
NCH = BPW // CB
G = 80
NG = (CB * S) // G
L = 16
NVR = D // L

_mesh = plsc.VectorSubcoreMesh(core_axis_name="c", subcore_axis_name="s")

V = 1000000
TBLK = 128
NBLK = V // TBLK
VTAIL = NBLK * TBLK
BLK_PER_W = NBLK // NW
BLK_REM = NBLK % NW


@functools.partial(
    pl.kernel,
    mesh=_mesh,
    out_type=jax.ShapeDtypeStruct((V * D // 2,), jnp.int32),
    scratch_types=[
        pltpu.VMEM((2 * D, TBLK), jnp.float32),
        pltpu.VMEM((2 * TBLK * D // 2,), jnp.int32),
        pltpu.VMEM(((V - VTAIL) * D // 2,), jnp.int32),
        pltpu.SemaphoreType.DMA((2,)),
        pltpu.SemaphoreType.DMA((2,)),
    ],
    compiler_params=pltpu.CompilerParams(
        use_tc_tiling_on_sc=True, needs_layout_passes=False
    ),
)
def _table_lin(tt_hbm, tail_hbm, out_hbm, slab_v, tslab_v, tail_v, in_sem, out_sem):
    wid = lax.axis_index("s") * NC + lax.axis_index("c")
    base = wid * BLK_PER_W + jnp.minimum(wid, BLK_REM)
    NPAIR = BLK_PER_W // 2
    lane16 = lax.broadcasted_iota(jnp.int32, (L,), 0)
    cm = [lane16 + L * m for m in range(TBLK // L)]
    DW = D // 2
    c2 = [
        [(lane16 + L * m) * DW + s * TBLK * DW for m in range(TBLK // L)]
        for s in range(2)
    ]

    def in_copy(c, s):
        return pltpu.make_async_copy(
            tt_hbm.at[:, pl.ds(c * TBLK, TBLK)],
            slab_v.at[pl.ds(s * D, D), :],
            in_sem.at[s],
        )

    def out_copy(c, s):
        return pltpu.make_async_copy(
            tslab_v.at[pl.ds(s * TBLK * DW, TBLK * DW)],
            out_hbm.at[pl.ds(c * (TBLK * DW), TBLK * DW)],
            out_sem.at[s],
        )

    def transpose_slot(s):
        @plsc.parallel_loop(0, DW, unroll=8)
        def tf(fp0):
            fpq = jnp.bitwise_and(fp0 + lane16, DW - 1)
            fe = 2 * fpq + s * D
            for m in range(TBLK // L):
                a = plsc.load_gather(slab_v, [fe, cm[m]])
                b = plsc.load_gather(slab_v, [fe + 1, cm[m]])
                w = plsc.bitcast(
                    plsc.pack(a, b, format=plsc.PackFormat.INTERLEAVED),
                    jnp.int32,
                )
                plsc.store_scatter(tslab_v, [c2[s][m] + fpq], w)

    in_copy(base, 0).start()

    def pair_body(t, carry):
        c0 = base + 2 * t

        in_copy(c0 + 1, 1).start()
        in_copy(c0, 0).wait()

        @pl.when(t >= 1)
        def _():
            out_copy(c0 - 2, 0).wait()

        transpose_slot(0)
        out_copy(c0, 0).start()

        @pl.when(t + 1 < NPAIR)
        def _():
            in_copy(c0 + 2, 0).start()

        in_copy(c0 + 1, 1).wait()

        @pl.when(t >= 1)
        def _():
            out_copy(c0 - 1, 1).wait()

        transpose_slot(1)
        out_copy(c0 + 1, 1).start()
        return carry

    lax.fori_loop(0, NPAIR, pair_body, 0)

    out_copy(base + 2 * NPAIR - 2, 0).wait()
    out_copy(base + 2 * NPAIR - 1, 1).wait()

    @pl.when(wid < BLK_REM)
    def _():
        c = base + 2 * NPAIR
        in_copy(c, 0).start()
        in_copy(c, 0).wait()
        transpose_slot(0)
        out_copy(c, 0).start()
        out_copy(c, 0).wait()

    @pl.when(wid == 0)
    def _():
        pltpu.sync_copy(tail_hbm, tail_v)
        pltpu.sync_copy(
            tail_v, out_hbm.at[pl.ds(VTAIL * (D // 2), (V - VTAIL) * (D // 2))]
        )


@functools.partial(
    pl.kernel,
    mesh=_mesh,
    out_type=jax.ShapeDtypeStruct((B * D,), jnp.float32),
    scratch_types=[
        pltpu.VMEM((3, CB * S), jnp.int32),
        pltpu.VMEM((2, CB * S, D // 2), jnp.int32),
        pltpu.VMEM((2 * CB * D,), jnp.float32),
        pltpu.SemaphoreType.DMA((3,)),
        pltpu.SemaphoreType.DMA((2,)),
        pltpu.SemaphoreType.DMA((2,)),
    ],
    compiler_params=pltpu.CompilerParams(
        use_tc_tiling_on_sc=False, needs_layout_passes=False
    ),
)
def _emb_pool(
    x_hbm, table_hbm, out_hbm, idx_v, rows_v, out_v, idx_sem, gat_sem, out_sem
):
    wid = lax.axis_index("s") * NC + lax.axis_index("c")
    lane = lax.broadcasted_iota(jnp.int32, (L,), 0)
    lo8 = jnp.minimum(jnp.maximum(8 - lane, 0), 1)
    hi8 = 1 - lo8

    def idx_copy(c, sl):
        return pltpu.make_async_copy(
            x_hbm.at[pl.ds((wid * BPW + c * CB) * S, CB * S)],
            idx_v.at[sl],
            idx_sem.at[sl],
        )

    def fire_gathers(bsl, gsl):
        for j in range(NG):
            pltpu.async_copy(
                table_hbm.at[idx_v.at[bsl, pl.ds(j * G, G)]],
                rows_v.at[gsl, pl.ds(j * G, G)],
                gat_sem.at[gsl],
            )

    def wait_gathers(bsl, gsl):
        for j in range(NG):
            pltpu.make_async_copy(
                table_hbm.at[idx_v.at[bsl, pl.ds(j * G, G)]],
                rows_v.at[gsl, pl.ds(j * G, G)],
                gat_sem.at[gsl],
            ).wait()

    idx_copy(0, 0).start()
    idx_copy(0, 0).wait()
    fire_gathers(0, 0)
    idx_copy(1, 1).start()

    def chunk_body(i, carry):
        row0 = wid * BPW + i * CB
        cur = jnp.bitwise_and(i, 1)
        nxt = jnp.bitwise_and(i + 1, 1)
        bsl = lax.rem(i, 3)
        bsl1 = lax.rem(i + 1, 3)
        bsl2 = lax.rem(i + 2, 3)

        @pl.when(i + 1 < NCH)
        def _():
            idx_copy(i + 1, bsl1).wait()
            fire_gathers(bsl1, nxt)

        @pl.when(i + 2 < NCH)
        def _():
            idx_copy(i + 2, bsl2).start()

        invs = []
        for r in range(CB):
            if r % 2 == 0:
                full0 = (S * r) // L
                shared_k = full0 + 12
                shared_mask = lo8
            else:
                shared_k = (S * r - 8) // L
                full0 = shared_k + 1
                shared_mask = hi8
            ones = (
                jnp.minimum(idx_v[bsl, pl.ds(L * shared_k, L)], 1) * shared_mask
            )
            for k in range(full0, full0 + 12):
                ones = ones + jnp.minimum(idx_v[bsl, pl.ds(L * k, L)], 1)
            cnt = ones[0]
            for j in range(1, L):
                cnt = cnt + ones[j]
            denom = jnp.maximum(
                jnp.broadcast_to(cnt, (L,)).astype(jnp.float32), 1.0
            )
            invs.append(1.0 / denom)

        wait_gathers(bsl, cur)

        @pl.when(i >= 2)
        def _():
            pltpu.make_async_copy(
                out_v.at[pl.ds(cur * (CB * D), CB * D)],
                out_hbm.at[pl.ds((row0 - 2 * CB) * D, CB * D)],
                out_sem.at[cur],
            ).wait()

        for r in range(CB):
            def srow(s, accs):
                row = r * S + s
                w0 = rows_v[cur, row, pl.ds(0, L)]
                w1 = rows_v[cur, row, pl.ds(L, L)]
                e0, o0 = plsc.unpack(
                    plsc.bitcast(w0, jnp.bfloat16),
                    format=plsc.PackFormat.INTERLEAVED,
                )
                e1, o1 = plsc.unpack(
                    plsc.bitcast(w1, jnp.bfloat16),
                    format=plsc.PackFormat.INTERLEAVED,
                )
                return (accs[0] + e0, accs[1] + o0, accs[2] + e1, accs[3] + o1)

            accs = lax.fori_loop(
                0,
                S,
                srow,
                tuple(jnp.zeros((L,), jnp.float32) for _ in range(NVR)),
                unroll=8,
            )
            obase = cur * (CB * D) + r * D + 2 * lane
            plsc.store_scatter(out_v, [obase], accs[0] * invs[r])
            plsc.store_scatter(out_v, [obase + 1], accs[1] * invs[r])
            plsc.store_scatter(out_v, [obase + D // 2], accs[2] * invs[r])
            plsc.store_scatter(out_v, [obase + D // 2 + 1], accs[3] * invs[r])

        pltpu.async_copy(
            out_v.at[pl.ds(cur * (CB * D), CB * D)],
            out_hbm.at[pl.ds(row0 * D, CB * D)],
            out_sem.at[cur],
        )
        return carry

    lax.fori_loop(0, NCH, chunk_body, 0)

    for t in (NCH - 2, NCH - 1):
        pltpu.make_async_copy(
            out_v.at[pl.ds((t % 2) * (CB * D), CB * D)],
            out_hbm.at[pl.ds((wid * BPW + t * CB) * D, CB * D)],
            out_sem.at[t % 2],
        ).wait()


def kernel(x, table):
    tail = lax.bitcast_convert_type(
        table[VTAIL:].astype(jnp.bfloat16).reshape(-1, 2), jnp.int32
    )
    tlin = _table_lin(table.T, tail)
    out = _emb_pool(x.reshape(-1), tlin.reshape(V, D // 2))
    return out.reshape(B, D)


# speedup vs baseline: 1.4735x; 1.0218x over previous
import functools

import jax
import jax.numpy as jnp
from jax import lax
from jax.experimental import pallas as pl
from jax.experimental.pallas import tpu as pltpu
from jax.experimental.pallas import tpu_sc as plsc

B = 16384
S = 200
D = 64
NC = 2
NS = 16
NW = NC * NS
BPW = B // NW
CB = 8
NCH = BPW // CB
G = 80
NG = (CB * S) // G
L = 16
NVR = D // L

_mesh = plsc.VectorSubcoreMesh(core_axis_name="c", subcore_axis_name="s")

V = 1000000
TBLK = 128
NBLK = V // TBLK
VTAIL = NBLK * TBLK
BLK_PER_W = NBLK // NW
BLK_REM = NBLK % NW


@functools.partial(
    pl.kernel,
    mesh=_mesh,
    out_type=jax.ShapeDtypeStruct((V * D // 2,), jnp.int32),
    scratch_types=[
        pltpu.VMEM((2 * D, TBLK), jnp.float32),
        pltpu.VMEM((2 * TBLK * D // 2,), jnp.int32),
        pltpu.VMEM(((V - VTAIL) * D // 2,), jnp.int32),
        pltpu.SemaphoreType.DMA((2,)),
        pltpu.SemaphoreType.DMA((2,)),
    ],
    compiler_params=pltpu.CompilerParams(
        use_tc_tiling_on_sc=True, needs_layout_passes=False
    ),
)
def _table_lin(tt_hbm, tail_hbm, out_hbm, slab_v, tslab_v, tail_v, in_sem, out_sem):
    wid = lax.axis_index("s") * NC + lax.axis_index("c")
    base = wid * BLK_PER_W + jnp.minimum(wid, BLK_REM)
    NPAIR = BLK_PER_W // 2
    lane16 = lax.broadcasted_iota(jnp.int32, (L,), 0)
    cm = [lane16 + L * m for m in range(TBLK // L)]
    DW = D // 2
    c2 = [
        [(lane16 + L * m) * DW + s * TBLK * DW for m in range(TBLK // L)]
        for s in range(2)
    ]

    def in_copy(c, s):
        return pltpu.make_async_copy(
            tt_hbm.at[:, pl.ds(c * TBLK, TBLK)],
            slab_v.at[pl.ds(s * D, D), :],
            in_sem.at[s],
        )

    def out_copy(c, s):
        return pltpu.make_async_copy(
            tslab_v.at[pl.ds(s * TBLK * DW, TBLK * DW)],
            out_hbm.at[pl.ds(c * (TBLK * DW), TBLK * DW)],
            out_sem.at[s],
        )

    def transpose_slot(s):
        @plsc.parallel_loop(0, DW, unroll=8)
        def tf(fp0):
            fpq = jnp.bitwise_and(fp0 + lane16, DW - 1)
            fe = 2 * fpq + s * D
            for m in range(TBLK // L):
                a = plsc.load_gather(slab_v, [fe, cm[m]])
                b = plsc.load_gather(slab_v, [fe + 1, cm[m]])
                w = plsc.bitcast(
                    plsc.pack(a, b, format=plsc.PackFormat.INTERLEAVED),
                    jnp.int32,
                )
                plsc.store_scatter(tslab_v, [c2[s][m] + fpq], w)

    in_copy(base, 0).start()

    def pair_body(t, carry):
        c0 = base + 2 * t

        in_copy(c0 + 1, 1).start()
        in_copy(c0, 0).wait()

        @pl.when(t >= 1)
        def _():
            out_copy(c0 - 2, 0).wait()

        transpose_slot(0)
        out_copy(c0, 0).start()

        @pl.when(t + 1 < NPAIR)
        def _():
            in_copy(c0 + 2, 0).start()

        in_copy(c0 + 1, 1).wait()

        @pl.when(t >= 1)
        def _():
            out_copy(c0 - 1, 1).wait()

        transpose_slot(1)
        out_copy(c0 + 1, 1).start()
        return carry

    lax.fori_loop(0, NPAIR, pair_body, 0)

    out_copy(base + 2 * NPAIR - 2, 0).wait()
    out_copy(base + 2 * NPAIR - 1, 1).wait()

    @pl.when(wid < BLK_REM)
    def _():
        c = base + 2 * NPAIR
        in_copy(c, 0).start()
        in_copy(c, 0).wait()
        transpose_slot(0)
        out_copy(c, 0).start()
        out_copy(c, 0).wait()

    @pl.when(wid == 0)
    def _():
        pltpu.sync_copy(tail_hbm, tail_v)
        pltpu.sync_copy(
            tail_v, out_hbm.at[pl.ds(VTAIL * (D // 2), (V - VTAIL) * (D // 2))]
        )


@functools.partial(
    pl.kernel,
    mesh=_mesh,
    out_type=jax.ShapeDtypeStruct((B * D,), jnp.float32),
    scratch_types=[
        pltpu.VMEM((3, CB * S), jnp.int32),
        pltpu.VMEM((2, CB * S, D // 2), jnp.int32),
        pltpu.VMEM((2 * CB * D,), jnp.float32),
        pltpu.SemaphoreType.DMA((3,)),
        pltpu.SemaphoreType.DMA((2,)),
        pltpu.SemaphoreType.DMA((2,)),
    ],
    compiler_params=pltpu.CompilerParams(
        use_tc_tiling_on_sc=False, needs_layout_passes=False
    ),
)
def _emb_pool(
    x_hbm, table_hbm, out_hbm, idx_v, rows_v, out_v, idx_sem, gat_sem, out_sem
):
    wid = lax.axis_index("s") * NC + lax.axis_index("c")
    lane = lax.broadcasted_iota(jnp.int32, (L,), 0)
    lo8 = jnp.minimum(jnp.maximum(8 - lane, 0), 1)
    hi8 = 1 - lo8

    def idx_copy(c, sl):
        return pltpu.make_async_copy(
            x_hbm.at[pl.ds((wid * BPW + c * CB) * S, CB * S)],
            idx_v.at[sl],
            idx_sem.at[sl],
        )

    def fire_gathers(bsl, gsl):
        for j in range(NG):
            pltpu.async_copy(
                table_hbm.at[idx_v.at[bsl, pl.ds(j * G, G)]],
                rows_v.at[gsl, pl.ds(j * G, G)],
                gat_sem.at[gsl],
            )

    def wait_gathers(bsl, gsl):
        for j in range(NG):
            pltpu.make_async_copy(
                table_hbm.at[idx_v.at[bsl, pl.ds(j * G, G)]],
                rows_v.at[gsl, pl.ds(j * G, G)],
                gat_sem.at[gsl],
            ).wait()

    idx_copy(0, 0).start()
    idx_copy(0, 0).wait()
    fire_gathers(0, 0)
    idx_copy(1, 1).start()

    def chunk_body(i, carry):
        row0 = wid * BPW + i * CB
        cur = jnp.bitwise_and(i, 1)
        nxt = jnp.bitwise_and(i + 1, 1)
        bsl = lax.rem(i, 3)
        bsl1 = lax.rem(i + 1, 3)
        bsl2 = lax.rem(i + 2, 3)

        @pl.when(i + 1 < NCH)
        def _():
            idx_copy(i + 1, bsl1).wait()
            fire_gathers(bsl1, nxt)

        @pl.when(i + 2 < NCH)
        def _():
            idx_copy(i + 2, bsl2).start()

        invs = []
        for r in range(CB):
            if r % 2 == 0:
                full0 = (S * r) // L
                shared_k = full0 + 12
                shared_mask = lo8
            else:
                shared_k = (S * r - 8) // L
                full0 = shared_k + 1
                shared_mask = hi8
            ones = (
                jnp.minimum(idx_v[bsl, pl.ds(L * shared_k, L)], 1) * shared_mask
            )
            for k in range(full0, full0 + 12):
                ones = ones + jnp.minimum(idx_v[bsl, pl.ds(L * k, L)], 1)
            cnt = ones[0]
            for j in range(1, L):
                cnt = cnt + ones[j]
            denom = jnp.maximum(
                jnp.broadcast_to(cnt, (L,)).astype(jnp.float32), 1.0
            )
            invs.append(1.0 / denom)

        wait_gathers(bsl, cur)

        @pl.when(i >= 2)
        def _():
            pltpu.make_async_copy(
                out_v.at[pl.ds(cur * (CB * D), CB * D)],
                out_hbm.at[pl.ds((row0 - 2 * CB) * D, CB * D)],
                out_sem.at[cur],
            ).wait()

        for r in range(CB):
            def srow(s, accs):
                row = r * S + s
                w0 = rows_v[cur, row, pl.ds(0, L)]
                w1 = rows_v[cur, row, pl.ds(L, L)]
                e0, o0 = plsc.unpack(
                    plsc.bitcast(w0, jnp.bfloat16),
                    format=plsc.PackFormat.INTERLEAVED,
                )
                e1, o1 = plsc.unpack(
                    plsc.bitcast(w1, jnp.bfloat16),
                    format=plsc.PackFormat.INTERLEAVED,
                )
                return (accs[0] + e0, accs[1] + o0, accs[2] + e1, accs[3] + o1)

            accs = lax.fori_loop(
                0,
                S,
                srow,
                tuple(jnp.zeros((L,), jnp.float32) for _ in range(NVR)),
                unroll=8,
            )
            obase = cur * (CB * D) + r * D + 2 * lane
            plsc.store_scatter(out_v, [obase], accs[0] * invs[r])
            plsc.store_scatter(out_v, [obase + 1], accs[1] * invs[r])
            plsc.store_scatter(out_v, [obase + D // 2], accs[2] * invs[r])
            plsc.store_scatter(out_v, [obase + D // 2 + 1], accs[3] * invs[r])

        pltpu.async_copy(
            out_v.at[pl.ds(cur * (CB * D), CB * D)],
            out_hbm.at[pl.ds(row0 * D, CB * D)],
            out_sem.at[cur],
        )
        return carry

    lax.fori_loop(0, NCH, chunk_body, 0)

    for t in (NCH - 2, NCH - 1):
        pltpu.make_async_copy(
            out_v.at[pl.ds((t % 2) * (CB * D), CB * D)],
            out_hbm.at[pl.ds((wid * BPW + t * CB) * D, CB * D)],
            out_sem.at[t % 2],
        ).wait()


def kernel(x, table):
    tail = lax.bitcast_convert_type(
        table[VTAIL:].astype(jnp.bfloat16).reshape(-1, 2), jnp.int32
    )
    tlin = _table_lin(table.T, tail)
    out = _emb_pool(x.reshape(-1), tlin.reshape(V, D // 2))
    return out.reshape(B, D)
